# baseline (speedup 1.0000x reference)
import jax
import jax.numpy as jnp
from jax import lax
from jax.experimental import pallas as pl
from jax.experimental.pallas import tpu as pltpu
from jax.experimental.pallas import tpu_sc as plsc

NC = 2
NS = 16

B = 4096
L = 50
V = 1024
H = 64

NQ = 4
N_ITEMS = L * NQ * 2
ITEMS_PER_TILE = N_ITEMS // NS


def _prep_body(ytT_ref, ptT_ref, ttT_ref, yiT_ref, piT_ref, tab_ref, ids_ref):
    a = ytT_ref[...] + ttT_ref[:, 0:1]
    b = ptT_ref[...] + ttT_ref[:, 1:2]
    tab_ref[pl.ds(0, V * H)] = a.reshape(V * H)
    tab_ref[pl.ds(V * H, V * H)] = b.reshape(V * H)
    ids_ref[pl.ds(0, L)] = yiT_ref[...].reshape(L, B // 128, 128)
    ids_ref[pl.ds(L, L)] = piT_ref[...].reshape(L, B // 128, 128)


def _sc_body(tab_hbm, ids_hbm, out_hbm, tab_v, ids_v, obuf, sem, sem_i):
    t = lax.axis_index("c")
    sid = lax.axis_index("s")

    pltpu.sync_copy(tab_hbm.at[pl.ds(t * (V * H), V * H)], tab_v)

    def fire_ids(k_next):
        item_n = k_next * NS + sid

        @pl.when(item_n < N_ITEMS)
        def _():
            l_n = item_n // 8
            q_n = (item_n % 8) // 2
            pltpu.async_copy(
                ids_hbm.at[t * L + l_n, pl.ds(8 * q_n, 8)],
                ids_v.at[k_next % 2],
                sem_i,
            )

    def drain_piece():
        pltpu.make_async_copy(
            out_hbm.at[0, 0, pl.ds(0, 2), pl.ds(0, 8)], obuf.at[0], sem
        ).wait()

    fire_ids(0)

    def item_body(k, carry):
        item = k * NS + sid
        l = item // 8
        q = (item % 8) // 2
        rh = item % 2
        islot = k % 2
        fire_ids(k + 1)
        pltpu.make_async_copy(
            ids_hbm.at[0, pl.ds(0, 8)], ids_v.at[0], sem_i
        ).wait()

        def r_body(rp, c2):
            p = k * 2 + rp
            slot = p % 3

            @pl.when(p >= 2)
            def _():
                drain_piece()

            for ri in range(2):
                r = rh * 4 + rp * 2 + ri
                hbase = r * 8 * V
                for cl in range(8):
                    iv = [ids_v[islot, cl, pl.ds(16 * j, 16)] for j in range(8)]
                    for hh in range(0, 8, 2):
                        vecs = [
                            plsc.load_gather(tab_v, [iv[j] + (hbase + h * V)])
                            for h in (hh, hh + 1)
                            for j in range(8)
                        ]
                        for jj, h in ((0, hh), (8, hh + 1)):
                            for j in range(8):
                                obuf[slot, ri, cl, h, pl.ds(16 * j, 16)] = vecs[jj + j]
            pltpu.async_copy(
                obuf.at[slot],
                out_hbm.at[t, l, pl.ds(rh * 4 + rp * 2, 2), pl.ds(8 * q, 8)],
                sem,
            )
            return c2

        lax.fori_loop(0, 2, r_body, 0)

        return carry

    lax.fori_loop(0, ITEMS_PER_TILE, item_body, 0)
    for _ in range(2):
        drain_piece()


def kernel(yaw_ids, pitch_ids, yaw_table, pitch_table, type_table):
    tab, ids = pl.pallas_call(
        _prep_body,
        out_shape=(
            jax.ShapeDtypeStruct((2 * V * H,), jnp.float32),
            jax.ShapeDtypeStruct((2 * L, B // 128, 128), jnp.int32),
        ),
    )(
        jnp.swapaxes(yaw_table, 0, 1),
        jnp.swapaxes(pitch_table, 0, 1),
        jnp.swapaxes(type_table, 0, 1),
        jnp.swapaxes(yaw_ids, 0, 1).astype(jnp.int32),
        jnp.swapaxes(pitch_ids, 0, 1).astype(jnp.int32),
    )

    mesh = plsc.VectorSubcoreMesh(
        core_axis_name="c", subcore_axis_name="s", num_cores=NC, num_subcores=NS
    )
    out6 = pl.kernel(
        _sc_body,
        out_type=jax.ShapeDtypeStruct((2, L, H // 8, B // 128, 8, 128), jnp.float32),
        mesh=mesh,
        scratch_types=[
            pltpu.VMEM((V * H,), jnp.float32),
            pltpu.VMEM((2, 8, 128), jnp.int32),
            pltpu.VMEM((3, 2, 8, 8, 128), jnp.float32),
            pltpu.SemaphoreType.DMA,
            pltpu.SemaphoreType.DMA,
        ],
        compiler_params=pltpu.CompilerParams(
            use_tc_tiling_on_sc=True, needs_layout_passes=False
        ),
    )(tab, ids)

    return jnp.transpose(out6, (3, 5, 0, 1, 2, 4)).reshape(B, 2, L, H)


# speedup vs baseline: 1.8318x; 1.8318x over previous
import jax
import jax.numpy as jnp
from jax import lax
from jax.experimental import pallas as pl
from jax.experimental.pallas import tpu as pltpu
from jax.experimental.pallas import tpu_sc as plsc

NC = 2
NS = 16

B = 4096
L = 50
V = 1024
H = 64

NQ = 4
N_ITEMS = L * NQ * 2
ITEMS_PER_TILE = N_ITEMS // NS


def _prep_body(ytT_ref, ptT_ref, ttT_ref, yiT_ref, piT_ref, tab_ref, ids_ref):
    a = ytT_ref[...] + ttT_ref[:, 0:1]
    b = ptT_ref[...] + ttT_ref[:, 1:2]
    tab_ref[pl.ds(0, V * H)] = a.reshape(V * H)
    tab_ref[pl.ds(V * H, V * H)] = b.reshape(V * H)
    ids_ref[pl.ds(0, L)] = yiT_ref[...].reshape(L, B // 128, 128)
    ids_ref[pl.ds(L, L)] = piT_ref[...].reshape(L, B // 128, 128)


def _sc_body(tab_hbm, ids_hbm, out_hbm, tab_v, ids_v, obuf, sem, sem_i):
    t = lax.axis_index("c")
    sid = lax.axis_index("s")

    pltpu.sync_copy(tab_hbm.at[pl.ds(t * (V * H), V * H)], tab_v)

    def fire_ids(k_next):
        item_n = k_next * NS + sid

        @pl.when(item_n < N_ITEMS)
        def _():
            l_n = item_n // 8
            q_n = (item_n % 8) // 2
            pltpu.async_copy(
                ids_hbm.at[t * L + l_n, pl.ds(8 * q_n, 8)],
                ids_v.at[k_next % 2],
                sem_i,
            )

    def drain_piece():
        pltpu.make_async_copy(
            out_hbm.at[0, 0, 0, pl.ds(0, 8)], obuf.at[0], sem
        ).wait()

    fire_ids(0)

    def item_body(k, carry):
        item = k * NS + sid
        l = item // 8
        q = (item % 8) // 2
        rh = item % 2
        islot = k % 2
        fire_ids(k + 1)
        pltpu.make_async_copy(
            ids_hbm.at[0, pl.ds(0, 8)], ids_v.at[0], sem_i
        ).wait()

        def r_body(rl, c2):
            r = rh * 4 + rl
            p = k * 4 + rl
            slot = p % 6

            @pl.when(p >= 5)
            def _():
                drain_piece()

            hbase = r * 8 * V
            for cl in range(8):
                iv = [ids_v[islot, cl, pl.ds(16 * j, 16)] for j in range(8)]
                for hh in range(0, 8, 2):
                    vecs = [
                        plsc.load_gather(tab_v, [iv[j] + (hbase + h * V)])
                        for h in (hh, hh + 1)
                        for j in range(8)
                    ]
                    for jj, h in ((0, hh), (8, hh + 1)):
                        for j in range(8):
                            obuf[slot, cl, h, pl.ds(16 * j, 16)] = vecs[jj + j]
            pltpu.async_copy(
                obuf.at[slot],
                out_hbm.at[t, l, r, pl.ds(8 * q, 8)],
                sem,
            )
            return c2

        lax.fori_loop(0, 4, r_body, 0)

        return carry

    lax.fori_loop(0, ITEMS_PER_TILE, item_body, 0)
    for _ in range(5):
        drain_piece()


def kernel(yaw_ids, pitch_ids, yaw_table, pitch_table, type_table):
    tab, ids = pl.pallas_call(
        _prep_body,
        out_shape=(
            jax.ShapeDtypeStruct((2 * V * H,), jnp.float32),
            jax.ShapeDtypeStruct((2 * L, B // 128, 128), jnp.int32),
        ),
    )(
        jnp.swapaxes(yaw_table, 0, 1),
        jnp.swapaxes(pitch_table, 0, 1),
        jnp.swapaxes(type_table, 0, 1),
        jnp.swapaxes(yaw_ids, 0, 1).astype(jnp.int32),
        jnp.swapaxes(pitch_ids, 0, 1).astype(jnp.int32),
    )

    mesh = plsc.VectorSubcoreMesh(
        core_axis_name="c", subcore_axis_name="s", num_cores=NC, num_subcores=NS
    )
    out6 = pl.kernel(
        _sc_body,
        out_type=jax.ShapeDtypeStruct((2, L, H // 8, B // 128, 8, 128), jnp.float32),
        mesh=mesh,
        scratch_types=[
            pltpu.VMEM((V * H,), jnp.float32),
            pltpu.VMEM((2, 8, 128), jnp.int32),
            pltpu.VMEM((6, 8, 8, 128), jnp.float32),
            pltpu.SemaphoreType.DMA,
            pltpu.SemaphoreType.DMA,
        ],
        compiler_params=pltpu.CompilerParams(
            use_tc_tiling_on_sc=True, needs_layout_passes=False
        ),
    )(tab, ids)

    return jnp.transpose(out6, (3, 5, 0, 1, 2, 4)).reshape(B, 2, L, H)
